# baseline (speedup 1.0000x reference)
import jax
import jax.numpy as jnp
from jax import lax
from jax.experimental import pallas as pl
from jax.experimental.pallas import tpu as pltpu
from jax.experimental.pallas import tpu_sc as plsc

VOCAB = 1000000
EMBED_DIM = 64
BATCH = 4096
HIST = 200

NUM_CORES = 2
NUM_SUBCORES = 16
NW = NUM_CORES * NUM_SUBCORES
SAMP_PER_W = BATCH // NW
HALF = SAMP_PER_W // 2

_COLS = tuple(range(0, HIST - 16, 16)) + (HIST - 16,)


PBLK = 800


def _pad_body(t_ref, out_ref):
    out_ref[...] = jnp.concatenate(
        [t_ref[...], jnp.zeros((PBLK, 128 - EMBED_DIM), jnp.float32)],
        axis=1)


def _pad_table(table):
    return pl.pallas_call(
        _pad_body,
        grid=(VOCAB // PBLK,),
        in_specs=[pl.BlockSpec((PBLK, EMBED_DIM), lambda i: (i, 0))],
        out_specs=pl.BlockSpec((PBLK, 128), lambda i: (i, 0)),
        out_shape=jax.ShapeDtypeStruct((VOCAB, 128), jnp.float32),
    )(table)


def _emb_body(x_hbm, table_hbm, out_hbm,
              xbuf, idx0, idx1, rows0, rows1, gsem0, gsem1, ssem0, ssem1):
    w = lax.axis_index("s") * NUM_CORES + lax.axis_index("c")
    s0 = w * SAMP_PER_W

    idxs = (idx0, idx1)
    bufs = (rows0, rows1)
    gsems = (gsem0, gsem1)
    ssems = (ssem0, ssem1)

    def start_gather(j, b):
        for c in _COLS:
            idxs[b][pl.ds(c, 16)] = xbuf[j % HALF, pl.ds(c, 16)]
        pltpu.async_copy(table_hbm.at[idxs[b]], bufs[b], gsems[b])

    def wait_gather(b):
        pltpu.make_async_copy(
            table_hbm.at[pl.ds(0, HIST)], bufs[b], gsems[b]).wait()

    def start_store(j, b):
        pltpu.async_copy(bufs[b], out_hbm.at[s0 + j], ssems[b])

    def wait_store(b):
        pltpu.make_async_copy(bufs[b], out_hbm.at[s0], ssems[b]).wait()

    def stage_x(half):
        pltpu.sync_copy(x_hbm.at[pl.ds(s0 + half * HALF, HALF)], xbuf)

    stage_x(0)
    start_gather(0, 0)
    wait_gather(0)
    start_gather(1, 1)
    start_store(0, 0)

    @pl.loop(0, (HALF - 2) // 2)
    def _pair(p):
        j = 1 + 2 * p
        for b in (1, 0):
            wait_gather(b)
            wait_store(1 - b)
            start_gather(j + 1, 1 - b)
            start_store(j, b)
            j = j + 1

    wait_gather(1)
    stage_x(1)
    wait_store(0)
    start_gather(HALF, 0)
    start_store(HALF - 1, 1)

    @pl.loop(0, (HALF - 2) // 2)
    def _pair2(p):
        j = HALF + 2 * p
        for b in (0, 1):
            wait_gather(b)
            wait_store(1 - b)
            start_gather(j + 1, 1 - b)
            start_store(j, b)
            j = j + 1

    wait_gather(0)
    wait_store(1)
    start_gather(SAMP_PER_W - 1, 1)
    start_store(SAMP_PER_W - 2, 0)
    wait_gather(1)
    wait_store(0)
    start_store(SAMP_PER_W - 1, 1)
    wait_store(1)


def _gather(x, table128):
    mesh = plsc.VectorSubcoreMesh(
        core_axis_name="c", subcore_axis_name="s",
        num_cores=NUM_CORES, num_subcores=NUM_SUBCORES)
    return pl.kernel(
        _emb_body,
        out_type=jax.ShapeDtypeStruct((BATCH, HIST, 128), jnp.float32),
        mesh=mesh,
        scratch_types=[
            pltpu.VMEM((HALF, HIST), jnp.int32),
            pltpu.VMEM((HIST,), jnp.int32),
            pltpu.VMEM((HIST,), jnp.int32),
            pltpu.VMEM((HIST, 128), jnp.float32),
            pltpu.VMEM((HIST, 128), jnp.float32),
            pltpu.SemaphoreType.DMA,
            pltpu.SemaphoreType.DMA,
            pltpu.SemaphoreType.DMA,
            pltpu.SemaphoreType.DMA,
        ],
        compiler_params=pltpu.CompilerParams(use_tc_tiling_on_sc=True),
    )(x, table128)


CBLK = 8


def _slice_body(in_ref, out_ref):
    out_ref[...] = in_ref[:, :, :EMBED_DIM]


def _slice_out(out128):
    return pl.pallas_call(
        _slice_body,
        grid=(BATCH // CBLK,),
        in_specs=[pl.BlockSpec((CBLK, HIST, 128), lambda i: (i, 0, 0))],
        out_specs=pl.BlockSpec((CBLK, HIST, EMBED_DIM), lambda i: (i, 0, 0)),
        out_shape=jax.ShapeDtypeStruct((BATCH, HIST, EMBED_DIM), jnp.float32),
    )(out128)


@jax.jit
def _embedding_sc(x, table):
    return _slice_out(_gather(x, _pad_table(table)))


def kernel(x, table):
    return _embedding_sc(x, table)


# speedup vs baseline: 1.9611x; 1.9611x over previous
import jax
import jax.numpy as jnp
from jax import lax
from jax.experimental import pallas as pl
from jax.experimental.pallas import tpu as pltpu
from jax.experimental.pallas import tpu_sc as plsc

VOCAB = 1000000
EMBED_DIM = 64
BATCH = 4096
HIST = 200

NUM_CORES = 2
NUM_SUBCORES = 16
NW = NUM_CORES * NUM_SUBCORES
SAMP_PER_W = BATCH // NW
HALF = SAMP_PER_W // 2

_COLS = tuple(range(0, HIST - 16, 16)) + (HIST - 16,)


def _emb_body(x_hbm, table_hbm, out_hbm,
              xbuf, idx0, idx1, rows0, rows1, data0, data1,
              gsem0, gsem1, ssem0, ssem1):
    w = lax.axis_index("s") * NUM_CORES + lax.axis_index("c")
    s0 = w * SAMP_PER_W

    idxs = (idx0, idx1)
    bufs = (rows0, rows1)
    dats = (data0, data1)
    gsems = (gsem0, gsem1)
    ssems = (ssem0, ssem1)

    def start_gather(j, b):
        for c in _COLS:
            idxs[b][pl.ds(c, 16)] = xbuf[j % HALF, pl.ds(c, 16)]
        pltpu.async_copy(table_hbm.at[idxs[b]], bufs[b], gsems[b])

    def wait_gather(b):
        pltpu.make_async_copy(
            table_hbm.at[pl.ds(0, HIST)], bufs[b], gsems[b]).wait()

    def start_store(j, b):
        @pl.loop(0, HIST)
        def _row(h):
            for c in range(0, EMBED_DIM, 16):
                dats[b][h, pl.ds(c, 16)] = bufs[b][h, pl.ds(c, 16)]
        pltpu.async_copy(dats[b], out_hbm.at[s0 + j], ssems[b])

    def wait_store(b):
        pltpu.make_async_copy(dats[b], out_hbm.at[s0], ssems[b]).wait()

    def stage_x(half):
        pltpu.sync_copy(x_hbm.at[pl.ds(s0 + half * HALF, HALF)], xbuf)

    stage_x(0)
    start_gather(0, 0)
    wait_gather(0)
    start_gather(1, 1)
    start_store(0, 0)

    @pl.loop(0, (HALF - 2) // 2)
    def _pair(p):
        j = 1 + 2 * p
        for b in (1, 0):
            wait_gather(b)
            wait_store(1 - b)
            start_gather(j + 1, 1 - b)
            start_store(j, b)
            j = j + 1

    wait_gather(1)
    stage_x(1)
    wait_store(0)
    start_gather(HALF, 0)
    start_store(HALF - 1, 1)

    @pl.loop(0, (HALF - 2) // 2)
    def _pair2(p):
        j = HALF + 2 * p
        for b in (0, 1):
            wait_gather(b)
            wait_store(1 - b)
            start_gather(j + 1, 1 - b)
            start_store(j, b)
            j = j + 1

    wait_gather(0)
    wait_store(1)
    start_gather(SAMP_PER_W - 1, 1)
    start_store(SAMP_PER_W - 2, 0)
    wait_gather(1)
    wait_store(0)
    start_store(SAMP_PER_W - 1, 1)
    wait_store(1)


@jax.jit
def _embedding_sc(x, table):
    table128 = jnp.pad(table, ((0, 0), (0, 128 - EMBED_DIM)))
    mesh = plsc.VectorSubcoreMesh(
        core_axis_name="c", subcore_axis_name="s",
        num_cores=NUM_CORES, num_subcores=NUM_SUBCORES)
    return pl.kernel(
        _emb_body,
        out_type=jax.ShapeDtypeStruct((BATCH, HIST, EMBED_DIM), jnp.float32),
        mesh=mesh,
        scratch_types=[
            pltpu.VMEM((HALF, HIST), jnp.int32),
            pltpu.VMEM((HIST,), jnp.int32),
            pltpu.VMEM((HIST,), jnp.int32),
            pltpu.VMEM((HIST, 128), jnp.float32),
            pltpu.VMEM((HIST, 128), jnp.float32),
            pltpu.VMEM((HIST, EMBED_DIM), jnp.float32),
            pltpu.VMEM((HIST, EMBED_DIM), jnp.float32),
            pltpu.SemaphoreType.DMA,
            pltpu.SemaphoreType.DMA,
            pltpu.SemaphoreType.DMA,
            pltpu.SemaphoreType.DMA,
        ],
        compiler_params=pltpu.CompilerParams(use_tc_tiling_on_sc=True),
    )(x, table128)


def kernel(x, table):
    return _embedding_sc(x, table)


# speedup vs baseline: 1.9667x; 1.0029x over previous
import jax
import jax.numpy as jnp
from jax import lax
from jax.experimental import pallas as pl
from jax.experimental.pallas import tpu as pltpu
from jax.experimental.pallas import tpu_sc as plsc

VOCAB = 1000000
EMBED_DIM = 64
BATCH = 4096
HIST = 200

NUM_CORES = 2
NUM_SUBCORES = 16
NW = NUM_CORES * NUM_SUBCORES
SAMP_PER_W = BATCH // NW
HALF = SAMP_PER_W // 2

_COLS = tuple(range(0, HIST - 16, 16)) + (HIST - 16,)


def _emb_body(x_hbm, table_hbm, out_hbm,
              xbuf, idx0, idx1, rows0, rows1, data0, data1,
              gsem0, gsem1, ssem0, ssem1):
    w = lax.axis_index("s") * NUM_CORES + lax.axis_index("c")
    s0 = w * SAMP_PER_W

    idxs = (idx0, idx1)
    bufs = (rows0, rows1)
    dats = (data0, data1)
    gsems = (gsem0, gsem1)
    ssems = (ssem0, ssem1)

    def start_gather(j, b):
        for c in _COLS:
            idxs[b][pl.ds(c, 16)] = xbuf[j % HALF, pl.ds(c, 16)]
        pltpu.async_copy(table_hbm.at[idxs[b]], bufs[b], gsems[b])

    def wait_gather(b):
        pltpu.make_async_copy(
            table_hbm.at[pl.ds(0, HIST)], bufs[b], gsems[b]).wait()

    def start_store(j, b):
        @pl.loop(0, HIST)
        def _row(h):
            for c in range(0, EMBED_DIM, 16):
                dats[b][h, pl.ds(c, 16)] = bufs[b][h, pl.ds(c, 16)]
        pltpu.async_copy(dats[b], out_hbm.at[s0 + j], ssems[b])

    def wait_store(b):
        pltpu.make_async_copy(dats[b], out_hbm.at[s0], ssems[b]).wait()

    def stage_x(half):
        pltpu.sync_copy(x_hbm.at[pl.ds(s0 + half * HALF, HALF)], xbuf)

    stage_x(0)
    start_gather(0, 0)
    wait_gather(0)
    start_gather(1, 1)
    start_store(0, 0)

    @pl.loop(0, (HALF - 2) // 2)
    def _pair(p):
        j = 1 + 2 * p
        for b in (1, 0):
            wait_gather(b)
            wait_store(1 - b)
            start_gather(j + 1, 1 - b)
            start_store(j, b)
            j = j + 1

    wait_gather(1)
    stage_x(1)
    wait_store(0)
    start_gather(HALF, 0)
    start_store(HALF - 1, 1)

    @pl.loop(0, (HALF - 2) // 2)
    def _pair2(p):
        j = HALF + 2 * p
        for b in (0, 1):
            wait_gather(b)
            wait_store(1 - b)
            start_gather(j + 1, 1 - b)
            start_store(j, b)
            j = j + 1

    wait_gather(0)
    wait_store(1)
    start_gather(SAMP_PER_W - 1, 1)
    start_store(SAMP_PER_W - 2, 0)
    wait_gather(1)
    wait_store(0)
    start_store(SAMP_PER_W - 1, 1)
    wait_store(1)


@jax.jit
def _embedding_sc(x, table):
    table128 = jnp.concatenate(
        [table, jnp.zeros((VOCAB, 128 - EMBED_DIM), jnp.float32)], axis=1)
    mesh = plsc.VectorSubcoreMesh(
        core_axis_name="c", subcore_axis_name="s",
        num_cores=NUM_CORES, num_subcores=NUM_SUBCORES)
    return pl.kernel(
        _emb_body,
        out_type=jax.ShapeDtypeStruct((BATCH, HIST, EMBED_DIM), jnp.float32),
        mesh=mesh,
        scratch_types=[
            pltpu.VMEM((HALF, HIST), jnp.int32),
            pltpu.VMEM((HIST,), jnp.int32),
            pltpu.VMEM((HIST,), jnp.int32),
            pltpu.VMEM((HIST, 128), jnp.float32),
            pltpu.VMEM((HIST, 128), jnp.float32),
            pltpu.VMEM((HIST, EMBED_DIM), jnp.float32),
            pltpu.VMEM((HIST, EMBED_DIM), jnp.float32),
            pltpu.SemaphoreType.DMA,
            pltpu.SemaphoreType.DMA,
            pltpu.SemaphoreType.DMA,
            pltpu.SemaphoreType.DMA,
        ],
        compiler_params=pltpu.CompilerParams(use_tc_tiling_on_sc=True),
    )(x, table128)


def kernel(x, table):
    return _embedding_sc(x, table)
